# baseline (speedup 1.0000x reference)
import jax
import jax.numpy as jnp
from jax import lax
from jax.experimental import pallas as pl
from jax.experimental.pallas import tpu as pltpu
from jax.experimental.pallas import tpu_sc as plsc

B = 4096
T = 200
D = 64
NW = 32
NTOK = B * T // NW
W = 100
SL = 4
CH = SL * W
NCH = NTOK // CH
NBUF = 2
LANES = 16


def _emb_body(idx_hbm, tok_hbm, pos_hbm, out_hbm,
              idx_v, pos_v, b0, b1, gs0, gs1, ws0, ws1):
    cid = lax.axis_index("c")
    sid = lax.axis_index("s")
    wid = sid * 2 + cid
    tok_base = wid * NTOK

    pltpu.sync_copy(idx_hbm.at[pl.ds(wid * NTOK, NTOK)], idx_v)
    pltpu.sync_copy(pos_hbm, pos_v)

    gb = [b0, b1]
    gs = [gs0, gs1]
    ws = [ws0, ws1]

    def g_start(c, j):
        pltpu.async_copy(tok_hbm.at[idx_v.at[pl.ds(CH * c, CH)]], gb[j], gs[j])

    def g_wait(c, j):
        pltpu.make_async_copy(tok_hbm.at[idx_v.at[pl.ds(CH * c, CH)]],
                              gb[j], gs[j]).wait()

    def w_start(c, j):
        row = tok_base + CH * c
        pltpu.async_copy(gb[j], out_hbm.at[pl.ds(row, CH)], ws[j])

    def w_wait(j):
        pltpu.make_async_copy(gb[j], out_hbm.at[pl.ds(0, CH)], ws[j]).wait()

    def add_pos(j):
        buf = gb[j]

        def t_body(t, carry):
            for half in range(2):
                r = t + half * T
                for cc in range(D // LANES):
                    s = pl.ds(LANES * cc, LANES)
                    buf[r, s] = buf[r, s] + pos_v[t, s]
            return carry

        lax.fori_loop(0, T, t_body, 0, unroll=2)

    for j in range(NBUF):
        g_start(j, j)

    def outer(i, carry):
        for j in range(NBUF):
            c = NBUF * i + j

            g_wait(c, j)
            add_pos(j)
            w_start(c, j)

            @pl.when(c + NBUF < NCH)
            def _():
                w_wait(j)
                g_start(c + NBUF, j)

        return carry

    lax.fori_loop(0, NCH // NBUF, outer, 0)

    for j in range(NBUF):
        w_wait(j)


@jax.jit
def _embed(idx2, tok, pos):
    kfn = pl.kernel(
        _emb_body,
        out_type=jax.ShapeDtypeStruct((B * T, D), jnp.float32),
        mesh=plsc.VectorSubcoreMesh(core_axis_name="c", subcore_axis_name="s"),
        compiler_params=pltpu.CompilerParams(use_tc_tiling_on_sc=False),
        scratch_types=[
            pltpu.VMEM((NTOK,), jnp.int32),
            pltpu.VMEM((T, D), jnp.float32),
            pltpu.VMEM((CH, D), jnp.float32),
            pltpu.VMEM((CH, D), jnp.float32),
            pltpu.SemaphoreType.DMA,
            pltpu.SemaphoreType.DMA,
            pltpu.SemaphoreType.DMA,
            pltpu.SemaphoreType.DMA,
        ],
    )
    return kfn(idx2, tok, pos)


def kernel(idx, token_embedding_table, position_embedding_table):
    idx2 = idx.astype(jnp.int32).reshape(B * T)
    out = _embed(idx2, token_embedding_table, position_embedding_table)
    return out.reshape(B, T, D)
